# baseline (speedup 1.0000x reference)
import functools

import jax
import jax.numpy as jnp
from jax import lax
from jax.experimental import pallas as pl
from jax.experimental.pallas import tpu as pltpu
from jax.experimental.pallas import tpu_sc as plsc

_NC = 2
_NS = 16
_NW = _NC * _NS
_LANES = 16

_SC_PARAMS = pltpu.CompilerParams(
    needs_layout_passes=False, use_tc_tiling_on_sc=False)


def _sc_degree(dst2, n_nodes):
  ept = dst2.shape[1]
  mesh = plsc.VectorSubcoreMesh(core_axis_name="c", subcore_axis_name="s")

  @functools.partial(
      pl.kernel,
      out_type=jax.ShapeDtypeStruct((_NW, n_nodes), jnp.float32),
      mesh=mesh,
      compiler_params=_SC_PARAMS,
      scratch_types=[
          pltpu.VMEM((ept,), jnp.int32),
          pltpu.VMEM((n_nodes,), jnp.float32),
      ],
  )
  def k(dst_hbm, out_hbm, dst_v, deg_v):
    c = lax.axis_index("c")
    s = lax.axis_index("s")
    wid = c * _NS + s
    pltpu.sync_copy(dst_hbm.at[wid], dst_v)

    def zero_body(i, carry):
      deg_v[pl.ds(pl.multiple_of(i * _LANES, 8), _LANES)] = jnp.zeros(
          (_LANES,), jnp.float32)
      return carry

    lax.fori_loop(0, n_nodes // _LANES, zero_body, 0)

    ones = jnp.ones((_LANES,), jnp.float32)

    def add_body(i, carry):
      idx = dst_v[pl.ds(pl.multiple_of(i * _LANES, 8), _LANES)]
      plsc.addupdate_scatter(deg_v, [idx], ones)
      return carry

    lax.fori_loop(0, ept // _LANES, add_body, 0)
    pltpu.sync_copy(deg_v, out_hbm.at[wid])

  return k(dst2)


def _sc_aggregate(g, src2, dst3, zeros, n_pad, k_depth, par=False):
  d = g.shape[1]
  nb, b = dst3.shape[1], dst3.shape[2]
  rpt = n_pad // _NS
  if k_depth:
    ngp = nb // (2 * k_depth)
    assert ngp * 2 * k_depth == nb
    kbuf = k_depth
  else:
    kbuf = 2 if par else 1
  mesh = plsc.VectorSubcoreMesh(core_axis_name="c", subcore_axis_name="s")

  @functools.partial(
      pl.kernel,
      out_type=jax.ShapeDtypeStruct((_NC, n_pad, d), jnp.float32),
      mesh=mesh,
      compiler_params=_SC_PARAMS,
      scratch_types=[
          pltpu.VMEM((nb * b,), jnp.int32),
          pltpu.VMEM((nb, b), jnp.int32),
          pltpu.VMEM((kbuf, b, d), jnp.float32),
          pltpu.VMEM((kbuf, b, d) if k_depth else (1, 1, 1), jnp.float32),
          pltpu.VMEM_SHARED((n_pad, d), jnp.float32),
          pltpu.SemaphoreType.DMA,
          pltpu.SemaphoreType.DMA,
          pltpu.SemaphoreType.DMA,
          pltpu.SemaphoreType.DMA,
      ],
  )
  def k(g_hbm, src_hbm, dst_hbm, z_hbm, out_hbm, src_v, dst_v, bufs0, bufs1,
        acc, gs0, gs1, ss0, ss1):
    c = lax.axis_index("c")
    s = lax.axis_index("s")
    wid = c * _NS + s
    pltpu.sync_copy(z_hbm.at[pl.ds(s * rpt, rpt)], acc.at[pl.ds(s * rpt, rpt)])
    pltpu.sync_copy(src_hbm.at[wid], src_v)
    pltpu.sync_copy(dst_hbm.at[wid], dst_v)
    plsc.subcore_barrier()

    if not k_depth:
      if par:
        @plsc.parallel_loop(0, nb, 1, unroll=2)
        def _(i):
          off = pl.multiple_of(i * b, 8)
          buf = bufs0.at[i % 2]
          pltpu.sync_copy(g_hbm.at[src_v.at[pl.ds(off, b)]], buf)
          pltpu.sync_copy(buf, acc.at[dst_v.at[i]], add=True)
      else:
        def sbody(i, carry):
          off = pl.multiple_of(i * b, 8)
          pltpu.sync_copy(g_hbm.at[src_v.at[pl.ds(off, b)]], bufs0.at[0])
          pltpu.sync_copy(bufs0.at[0], acc.at[dst_v.at[i]], add=True)
          return carry

        lax.fori_loop(0, nb, sbody, 0)
      plsc.subcore_barrier()
      pltpu.sync_copy(acc.at[pl.ds(s * rpt, rpt)],
                      out_hbm.at[c, pl.ds(s * rpt, rpt)])
      return

    bufs = (bufs0, bufs1)
    gsems = (gs0, gs1)
    ssems = (ss0, ss1)

    def fire_gathers(setid, jbase):
      for t in range(k_depth):
        off = pl.multiple_of((jbase + t) * b, 8)
        pltpu.async_copy(g_hbm.at[src_v.at[pl.ds(off, b)]],
                         bufs[setid].at[t], gsems[setid])

    def wait_gathers(setid):
      for t in range(k_depth):
        pltpu.make_async_copy(g_hbm.at[src_v.at[pl.ds(0, b)]],
                              bufs[setid].at[t], gsems[setid]).wait()

    def fire_scatters(setid, jbase):
      for t in range(k_depth):
        pltpu.async_copy(bufs[setid].at[t], acc.at[dst_v.at[jbase + t]],
                         ssems[setid], add=True)

    def wait_scatters(setid):
      for t in range(k_depth):
        pltpu.make_async_copy(bufs[setid].at[t], acc.at[dst_v.at[0]],
                              ssems[setid]).wait()

    fire_gathers(0, 0)

    def body(i, carry):
      j0 = i * 2 * k_depth
      wait_gathers(0)

      @pl.when(i > 0)
      def _():
        wait_scatters(1)

      fire_gathers(1, j0 + k_depth)
      fire_scatters(0, j0)
      wait_gathers(1)
      wait_scatters(0)

      @pl.when(i < ngp - 1)
      def _():
        fire_gathers(0, j0 + 2 * k_depth)

      fire_scatters(1, j0 + k_depth)
      return carry

    lax.fori_loop(0, ngp, body, 0)
    wait_scatters(1)
    plsc.subcore_barrier()
    pltpu.sync_copy(acc.at[pl.ds(s * rpt, rpt)],
                    out_hbm.at[c, pl.ds(s * rpt, rpt)])

  return k(g, src2, dst3, zeros)


def _dinv_block(degpt_ref):
  return lax.rsqrt(1.0 + jnp.sum(degpt_ref[...], axis=1, keepdims=True))


def _tc1_body(x_ref, w_ref, degpt_ref, out_ref):
  h = jnp.dot(x_ref[...], w_ref[...], preferred_element_type=jnp.float32)
  out_ref[...] = h * _dinv_block(degpt_ref)


def _tc2_body(acc_ref, g1_ref, degpt_ref, b1_ref, w2_ref, out_ref):
  dinv = _dinv_block(degpt_ref)
  a = acc_ref[...]
  t = dinv * (a[0] + a[1] + g1_ref[...]) + b1_ref[...]
  t = jnp.maximum(t, 0.0)
  out_ref[...] = jnp.dot(t, w2_ref[...],
                         preferred_element_type=jnp.float32) * dinv


def _tc3_body(acc_ref, g2_ref, degpt_ref, b2_ref, out_ref):
  dinv = _dinv_block(degpt_ref)
  a = acc_ref[...]
  out_ref[...] = dinv * (a[0] + a[1] + g2_ref[...]) + b2_ref[...]


_BM = 1000


def _tc1(x, w1, degpt):
  n, f = x.shape
  h = w1.shape[1]
  return pl.pallas_call(
      _tc1_body,
      grid=(n // _BM,),
      in_specs=[
          pl.BlockSpec((_BM, f), lambda i: (i, 0)),
          pl.BlockSpec((f, h), lambda i: (0, 0)),
          pl.BlockSpec((_BM, _NW), lambda i: (i, 0)),
      ],
      out_specs=pl.BlockSpec((_BM, h), lambda i: (i, 0)),
      out_shape=jax.ShapeDtypeStruct((n, h), jnp.float32),
  )(x, w1, degpt)


def _tc2(acc1, g1, degpt, b1, w2):
  n, h = g1.shape
  co = w2.shape[1]
  return pl.pallas_call(
      _tc2_body,
      grid=(n // _BM,),
      in_specs=[
          pl.BlockSpec((2, _BM, h), lambda i: (0, i, 0)),
          pl.BlockSpec((_BM, h), lambda i: (i, 0)),
          pl.BlockSpec((_BM, _NW), lambda i: (i, 0)),
          pl.BlockSpec((1, h), lambda i: (0, 0)),
          pl.BlockSpec((h, co), lambda i: (0, 0)),
      ],
      out_specs=pl.BlockSpec((_BM, co), lambda i: (i, 0)),
      out_shape=jax.ShapeDtypeStruct((n, co), jnp.float32),
  )(acc1, g1, degpt, b1, w2)


def _tc3(acc2, g2, degpt, b2):
  n, co = g2.shape
  return pl.pallas_call(
      _tc3_body,
      grid=(n // _BM,),
      in_specs=[
          pl.BlockSpec((2, _BM, co), lambda i: (0, i, 0)),
          pl.BlockSpec((_BM, co), lambda i: (i, 0)),
          pl.BlockSpec((_BM, _NW), lambda i: (i, 0)),
          pl.BlockSpec((1, co), lambda i: (0, 0)),
      ],
      out_specs=pl.BlockSpec((_BM, co), lambda i: (i, 0)),
      out_shape=jax.ShapeDtypeStruct((n, co), jnp.float32),
  )(acc2, g2, degpt, b2)


def kernel(x, edge_index, W1, b1, W2, b2):
  n, f = x.shape
  e = edge_index.shape[1]
  src = edge_index[0].astype(jnp.int32)
  dst = edge_index[1].astype(jnp.int32)

  ept = e // _NW
  assert ept * _NW == e and n % _NS == 0 and n % _LANES == 0

  n_pad = ((n + 8 * _NS - 1) // (8 * _NS)) * (8 * _NS)

  src2 = src.reshape(_NW, ept)
  dst2 = dst.reshape(_NW, ept)

  def pad_indices(b, k_depth, even=False):
    nb = (ept + b - 1) // b
    if k_depth:
      gsz = 2 * k_depth
      nb = ((nb + gsz - 1) // gsz) * gsz
    elif even:
      nb = ((nb + 1) // 2) * 2
    pad = nb * b - ept
    src2p = jnp.concatenate(
        [src2, jnp.zeros((_NW, pad), jnp.int32)], axis=1)
    dst2p = jnp.concatenate(
        [dst2, jnp.full((_NW, pad), n_pad - 1, jnp.int32)], axis=1)
    return src2p, dst2p.reshape(_NW, nb, b)

  degp = _sc_degree(dst2, n)
  degpt = degp.T

  g1 = _tc1(x, W1, degpt)
  z128 = jnp.zeros((n_pad, f), jnp.float32)
  src1p, dst1p = pad_indices(b=96, k_depth=0)
  acc1 = _sc_aggregate(g1, src1p, dst1p, z128, n_pad, k_depth=0)
  g2 = _tc2(acc1, g1, degpt, b1.reshape(1, -1), W2)
  src2p, dst2p3 = pad_indices(b=128, k_depth=8)
  acc2 = _sc_aggregate(
      g2, src2p, dst2p3, jnp.zeros((n_pad, g2.shape[1]), jnp.float32),
      n_pad, k_depth=8)
  out = _tc3(acc2, g2, degpt, b2.reshape(1, -1))
  return out


# speedup vs baseline: 1.1993x; 1.1993x over previous
import functools

import jax
import jax.numpy as jnp
from jax import lax
from jax.experimental import pallas as pl
from jax.experimental.pallas import tpu as pltpu
from jax.experimental.pallas import tpu_sc as plsc

_NC = 2
_NS = 16
_NW = _NC * _NS
_LANES = 16

_SC_PARAMS = pltpu.CompilerParams(
    needs_layout_passes=False, use_tc_tiling_on_sc=False)


def _sc_degree(dst2, n_nodes):
  ept = dst2.shape[1]
  mesh = plsc.VectorSubcoreMesh(core_axis_name="c", subcore_axis_name="s")

  @functools.partial(
      pl.kernel,
      out_type=jax.ShapeDtypeStruct((_NW, n_nodes), jnp.float32),
      mesh=mesh,
      compiler_params=_SC_PARAMS,
      scratch_types=[
          pltpu.VMEM((ept,), jnp.int32),
          pltpu.VMEM((n_nodes,), jnp.float32),
      ],
  )
  def k(dst_hbm, out_hbm, dst_v, deg_v):
    c = lax.axis_index("c")
    s = lax.axis_index("s")
    wid = c * _NS + s
    pltpu.sync_copy(dst_hbm.at[wid], dst_v)

    def zero_body(i, carry):
      deg_v[pl.ds(pl.multiple_of(i * _LANES, 8), _LANES)] = jnp.zeros(
          (_LANES,), jnp.float32)
      return carry

    lax.fori_loop(0, n_nodes // _LANES, zero_body, 0)

    ones = jnp.ones((_LANES,), jnp.float32)

    def add_body(i, carry):
      idx = dst_v[pl.ds(pl.multiple_of(i * _LANES, 8), _LANES)]
      plsc.addupdate_scatter(deg_v, [idx], ones)
      return carry

    lax.fori_loop(0, ept // _LANES, add_body, 0)
    pltpu.sync_copy(deg_v, out_hbm.at[wid])

  return k(dst2)


def _sc_aggregate(g, src2, dst3, zeros, n_pad, k_depth):
  d = g.shape[1]
  nb, b = dst3.shape[1], dst3.shape[2]
  rpt = n_pad // _NS
  if k_depth:
    ngp = nb // (2 * k_depth)
    assert ngp * 2 * k_depth == nb
    kbuf = k_depth
  else:
    kbuf = 1
  mesh = plsc.VectorSubcoreMesh(core_axis_name="c", subcore_axis_name="s")

  @functools.partial(
      pl.kernel,
      out_type=jax.ShapeDtypeStruct((_NC, n_pad, d), jnp.float32),
      mesh=mesh,
      compiler_params=_SC_PARAMS,
      scratch_types=[
          pltpu.VMEM((nb * b,), jnp.int32),
          pltpu.VMEM((nb, b), jnp.int32),
          pltpu.VMEM((kbuf, b, d), jnp.float32),
          pltpu.VMEM((kbuf, b, d) if k_depth else (1, 1, 1), jnp.float32),
          pltpu.VMEM_SHARED((n_pad, d), jnp.float32),
          pltpu.SemaphoreType.DMA,
          pltpu.SemaphoreType.DMA,
          pltpu.SemaphoreType.DMA,
          pltpu.SemaphoreType.DMA,
      ],
  )
  def k(g_hbm, src_hbm, dst_hbm, z_hbm, out_hbm, src_v, dst_v, bufs0, bufs1,
        acc, gs0, gs1, ss0, ss1):
    c = lax.axis_index("c")
    s = lax.axis_index("s")
    wid = c * _NS + s
    pltpu.sync_copy(z_hbm.at[pl.ds(s * rpt, rpt)], acc.at[pl.ds(s * rpt, rpt)])
    pltpu.sync_copy(src_hbm.at[wid], src_v)
    pltpu.sync_copy(dst_hbm.at[wid], dst_v)
    plsc.subcore_barrier()

    if not k_depth:
      def sbody(i, carry):
        off = pl.multiple_of(i * b, 8)
        pltpu.sync_copy(g_hbm.at[src_v.at[pl.ds(off, b)]], bufs0.at[0])
        pltpu.sync_copy(bufs0.at[0], acc.at[dst_v.at[i]], add=True)
        return carry

      lax.fori_loop(0, nb, sbody, 0)
      plsc.subcore_barrier()
      pltpu.sync_copy(acc.at[pl.ds(s * rpt, rpt)],
                      out_hbm.at[c, pl.ds(s * rpt, rpt)])
      return

    bufs = (bufs0, bufs1)
    gsems = (gs0, gs1)
    ssems = (ss0, ss1)

    def fire_gathers(setid, jbase):
      for t in range(k_depth):
        off = pl.multiple_of((jbase + t) * b, 8)
        pltpu.async_copy(g_hbm.at[src_v.at[pl.ds(off, b)]],
                         bufs[setid].at[t], gsems[setid])

    def wait_gathers(setid):
      for t in range(k_depth):
        pltpu.make_async_copy(g_hbm.at[src_v.at[pl.ds(0, b)]],
                              bufs[setid].at[t], gsems[setid]).wait()

    def fire_scatters(setid, jbase):
      for t in range(k_depth):
        pltpu.async_copy(bufs[setid].at[t], acc.at[dst_v.at[jbase + t]],
                         ssems[setid], add=True)

    def wait_scatters(setid):
      for t in range(k_depth):
        pltpu.make_async_copy(bufs[setid].at[t], acc.at[dst_v.at[0]],
                              ssems[setid]).wait()

    fire_gathers(0, 0)

    def body(i, carry):
      j0 = i * 2 * k_depth
      wait_gathers(0)

      @pl.when(i > 0)
      def _():
        wait_scatters(1)

      fire_gathers(1, j0 + k_depth)
      fire_scatters(0, j0)
      wait_gathers(1)
      wait_scatters(0)

      @pl.when(i < ngp - 1)
      def _():
        fire_gathers(0, j0 + 2 * k_depth)

      fire_scatters(1, j0 + k_depth)
      return carry

    lax.fori_loop(0, ngp, body, 0)
    wait_scatters(1)
    plsc.subcore_barrier()
    pltpu.sync_copy(acc.at[pl.ds(s * rpt, rpt)],
                    out_hbm.at[c, pl.ds(s * rpt, rpt)])

  return k(g, src2, dst3, zeros)


def _dinv_block(degpt_ref):
  return lax.rsqrt(1.0 + jnp.sum(degpt_ref[...], axis=1, keepdims=True))


def _tc1_body(x_ref, w_ref, degpt_ref, out_ref):
  h = jnp.dot(x_ref[...], w_ref[...], preferred_element_type=jnp.float32)
  out_ref[...] = h * _dinv_block(degpt_ref)


def _tc2_body(acc_ref, g1_ref, degpt_ref, b1_ref, w2_ref, out_ref):
  dinv = _dinv_block(degpt_ref)
  a = acc_ref[...]
  t = dinv * (a[0] + a[1] + g1_ref[...]) + b1_ref[...]
  t = jnp.maximum(t, 0.0)
  out_ref[...] = jnp.dot(t, w2_ref[...],
                         preferred_element_type=jnp.float32) * dinv


def _tc3_body(acc_ref, g2_ref, degpt_ref, b2_ref, out_ref):
  dinv = _dinv_block(degpt_ref)
  a = acc_ref[...]
  out_ref[...] = dinv * (a[0] + a[1] + g2_ref[...]) + b2_ref[...]


_BM = 1000


def _tc1(x, w1, degpt):
  n, f = x.shape
  h = w1.shape[1]
  return pl.pallas_call(
      _tc1_body,
      grid=(n // _BM,),
      in_specs=[
          pl.BlockSpec((_BM, f), lambda i: (i, 0)),
          pl.BlockSpec((f, h), lambda i: (0, 0)),
          pl.BlockSpec((_BM, _NW), lambda i: (i, 0)),
      ],
      out_specs=pl.BlockSpec((_BM, h), lambda i: (i, 0)),
      out_shape=jax.ShapeDtypeStruct((n, h), jnp.float32),
  )(x, w1, degpt)


def _tc2(acc1, g1, degpt, b1, w2):
  n, h = g1.shape
  co = w2.shape[1]
  return pl.pallas_call(
      _tc2_body,
      grid=(n // _BM,),
      in_specs=[
          pl.BlockSpec((2, _BM, h), lambda i: (0, i, 0)),
          pl.BlockSpec((_BM, h), lambda i: (i, 0)),
          pl.BlockSpec((_BM, _NW), lambda i: (i, 0)),
          pl.BlockSpec((1, h), lambda i: (0, 0)),
          pl.BlockSpec((h, co), lambda i: (0, 0)),
      ],
      out_specs=pl.BlockSpec((_BM, co), lambda i: (i, 0)),
      out_shape=jax.ShapeDtypeStruct((n, co), jnp.float32),
  )(acc1, g1, degpt, b1, w2)


def _tc3(acc2, g2, degpt, b2):
  n, co = g2.shape
  return pl.pallas_call(
      _tc3_body,
      grid=(n // _BM,),
      in_specs=[
          pl.BlockSpec((2, _BM, co), lambda i: (0, i, 0)),
          pl.BlockSpec((_BM, co), lambda i: (i, 0)),
          pl.BlockSpec((_BM, _NW), lambda i: (i, 0)),
          pl.BlockSpec((1, co), lambda i: (0, 0)),
      ],
      out_specs=pl.BlockSpec((_BM, co), lambda i: (i, 0)),
      out_shape=jax.ShapeDtypeStruct((n, co), jnp.float32),
  )(acc2, g2, degpt, b2)


def kernel(x, edge_index, W1, b1, W2, b2):
  n, f = x.shape
  e = edge_index.shape[1]
  src = edge_index[0].astype(jnp.int32)
  dst = edge_index[1].astype(jnp.int32)

  ept = e // _NW
  assert ept * _NW == e and n % _NS == 0 and n % _LANES == 0

  n_pad = ((n + 8 * _NS - 1) // (8 * _NS)) * (8 * _NS)

  src2 = src.reshape(_NW, ept)
  dst2 = dst.reshape(_NW, ept)

  def pad_indices(b, k_depth):
    nb = (ept + b - 1) // b
    if k_depth:
      gsz = 2 * k_depth
      nb = ((nb + gsz - 1) // gsz) * gsz
    pad = nb * b - ept
    src2p = jnp.concatenate(
        [src2, jnp.zeros((_NW, pad), jnp.int32)], axis=1)
    dst2p = jnp.concatenate(
        [dst2, jnp.full((_NW, pad), n_pad - 1, jnp.int32)], axis=1)
    return src2p, dst2p.reshape(_NW, nb, b)

  degp = _sc_degree(dst2, n)
  degpt = degp.T

  g1 = _tc1(x, W1, degpt)
  z128 = jnp.zeros((n_pad, f), jnp.float32)
  src1p, dst1p = pad_indices(b=80, k_depth=0)
  acc1 = _sc_aggregate(g1, src1p, dst1p, z128, n_pad, k_depth=0)
  g2 = _tc2(acc1, g1, degpt, b1.reshape(1, -1), W2)
  src2p, dst2p3 = pad_indices(b=128, k_depth=8)
  acc2 = _sc_aggregate(
      g2, src2p, dst2p3, jnp.zeros((n_pad, g2.shape[1]), jnp.float32),
      n_pad, k_depth=8)
  out = _tc3(acc2, g2, degpt, b2.reshape(1, -1))
  return out
